# baseline (speedup 1.0000x reference)
import jax
import jax.numpy as jnp
from jax.experimental import pallas as pl
from jax.experimental.pallas import tpu as pltpu

_B, _N, _D, _H = 128, 8192, 64, 256
_ALPHA = 0.5
_TN = 4096
_NT = _N // _TN


def _cumsum_last(x):
  n = x.shape[-1]
  d = 1
  while d < n:
    x = x + jnp.concatenate([jnp.zeros_like(x[..., :d]), x[..., :-d]], axis=-1)
    d *= 2
  return x


def _resample_coeffs(wl, u0):
  m = jnp.max(wl, axis=-1, keepdims=True)
  e = jnp.exp(wl - m)
  s = jnp.sum(e, axis=-1, keepdims=True)
  probs = e / s
  soft = _ALPHA * probs + (1.0 - _ALPHA) / _N
  cum = _cumsum_last(soft)
  cum = cum / cum[..., -1:]
  t = cum * _N
  k = jnp.floor(t)
  g = k + jnp.where(k + u0 <= t, 1.0, 0.0)
  g = jnp.clip(g, 0.0, float(_N))
  gprev = jnp.concatenate([jnp.zeros_like(g[..., :1]), g[..., :-1]], axis=-1)
  count = g - gprev
  return count * (probs / (soft + 1e-8) + 1e-8)


def _body(wl_ref, u0_ref, p_ref, w1t_ref, w2_ref, o_ref, a_ref, acc_ref):
  j = pl.program_id(1)

  @pl.when(j == 0)
  def _():
    a_ref[...] = _resample_coeffs(wl_ref[0], u0_ref[0])
    acc_ref[...] = jnp.zeros_like(acc_ref)

  p = p_ref[0]
  z = jax.lax.dot_general(w1t_ref[...].astype(jnp.bfloat16),
                          p.astype(jnp.bfloat16),
                          (((1,), (1,)), ((), ())),
                          preferred_element_type=jnp.float32)
  h = jnp.tanh(z)
  lik = jnp.dot(w2_ref[...], h, preferred_element_type=jnp.float32)
  t_ = a_ref[:, pl.ds(j * _TN, _TN)] * jnp.exp(lik)
  est = jnp.dot(t_, p, preferred_element_type=jnp.float32)
  acc_ref[:, :_D] += est
  acc_ref[:, _D:] += jnp.sum(t_, axis=-1, keepdims=True)

  @pl.when(j == _NT - 1)
  def _():
    o_ref[...] = (acc_ref[:, :_D] / acc_ref[:, _D:])[None]


def kernel(particles, weights_log, u0, W1, w2):
  wl3 = weights_log.reshape(_B, 1, _N)
  u03 = u0.reshape(_B, 1, 1)
  w1t = W1.T
  w2r = w2.reshape(1, _H)

  est3 = pl.pallas_call(
      _body,
      grid=(_B, _NT),
      in_specs=[
          pl.BlockSpec((1, 1, _N), lambda b, j: (b, 0, 0)),
          pl.BlockSpec((1, 1, 1), lambda b, j: (b, 0, 0)),
          pl.BlockSpec((1, _TN, _D), lambda b, j: (b, j, 0)),
          pl.BlockSpec((_H, _D), lambda b, j: (0, 0)),
          pl.BlockSpec((1, _H), lambda b, j: (0, 0)),
      ],
      out_specs=pl.BlockSpec((1, 1, _D), lambda b, j: (b, 0, 0)),
      out_shape=jax.ShapeDtypeStruct((_B, 1, _D), jnp.float32),
      scratch_shapes=[
          pltpu.VMEM((1, _N), jnp.float32),
          pltpu.VMEM((1, _D + 1), jnp.float32),
      ],
      compiler_params=pltpu.CompilerParams(
          dimension_semantics=("arbitrary", "arbitrary")),
  )(wl3, u03, particles, w1t, w2r)

  return est3.reshape(_B, _D)


# speedup vs baseline: 1.0897x; 1.0897x over previous
import jax
import jax.numpy as jnp
from jax.experimental import pallas as pl
from jax.experimental.pallas import tpu as pltpu

_B, _N, _D, _H = 128, 8192, 64, 256
_ALPHA = 0.5
_CN = 4096


def _cumsum_last(x):
  n = x.shape[-1]
  d = 1
  while d < n:
    x = x + jnp.concatenate([jnp.zeros_like(x[..., :d]), x[..., :-d]], axis=-1)
    d *= 2
  return x


def _resample_coeffs(wl, u0):
  m = jnp.max(wl, axis=-1, keepdims=True)
  e = jnp.exp(wl - m)
  s = jnp.sum(e, axis=-1, keepdims=True)
  probs = e / s
  soft = _ALPHA * probs + (1.0 - _ALPHA) / _N
  cum = _cumsum_last(soft)
  cum = cum / cum[..., -1:]
  t = cum * _N
  k = jnp.floor(t)
  g = k + jnp.where(k + u0 <= t, 1.0, 0.0)
  g = jnp.clip(g, 0.0, float(_N))
  gprev = jnp.concatenate([jnp.zeros_like(g[..., :1]), g[..., :-1]], axis=-1)
  count = g - gprev
  return count * (probs / (soft + 1e-8) + 1e-8)


def _body(wl_ref, u0_ref, hbm_ref, w1t_ref, w2_ref, o_ref, buf_ref, sem_ref):
  b = pl.program_id(0)
  slot = jax.lax.rem(b, 2)
  nslot = jax.lax.rem(b + 1, 2)

  @pl.when(b == 0)
  def _():
    pltpu.make_async_copy(hbm_ref.at[0], buf_ref.at[0], sem_ref.at[0]).start()

  @pl.when(b + 1 < _B)
  def _():
    pltpu.make_async_copy(hbm_ref.at[b + 1], buf_ref.at[nslot],
                          sem_ref.at[nslot]).start()

  a = _resample_coeffs(wl_ref[0], u0_ref[0])

  pltpu.make_async_copy(hbm_ref.at[b], buf_ref.at[slot], sem_ref.at[slot]).wait()

  w1tb = w1t_ref[...].astype(jnp.bfloat16)
  est = jnp.zeros((1, _D), jnp.float32)
  zsum = jnp.zeros((1, 1), jnp.float32)
  for c in range(_N // _CN):
    p = buf_ref[slot, c * _CN:(c + 1) * _CN, :]
    z = jax.lax.dot_general(w1tb, p.astype(jnp.bfloat16),
                            (((1,), (1,)), ((), ())),
                            preferred_element_type=jnp.float32)
    h = jnp.tanh(z)
    lik = jnp.dot(w2_ref[...], h, preferred_element_type=jnp.float32)
    t_ = a[:, c * _CN:(c + 1) * _CN] * jnp.exp(lik)
    est += jnp.dot(t_, p, preferred_element_type=jnp.float32)
    zsum += jnp.sum(t_, axis=-1, keepdims=True)
  o_ref[...] = (est / zsum)[None]


def kernel(particles, weights_log, u0, W1, w2):
  wl3 = weights_log.reshape(_B, 1, _N)
  u03 = u0.reshape(_B, 1, 1)
  w1t = W1.T
  w2r = w2.reshape(1, _H)

  est3 = pl.pallas_call(
      _body,
      grid=(_B,),
      in_specs=[
          pl.BlockSpec((1, 1, _N), lambda b: (b, 0, 0)),
          pl.BlockSpec((1, 1, 1), lambda b: (b, 0, 0)),
          pl.BlockSpec(memory_space=pl.ANY),
          pl.BlockSpec((_H, _D), lambda b: (0, 0)),
          pl.BlockSpec((1, _H), lambda b: (0, 0)),
      ],
      out_specs=pl.BlockSpec((1, 1, _D), lambda b: (b, 0, 0)),
      out_shape=jax.ShapeDtypeStruct((_B, 1, _D), jnp.float32),
      scratch_shapes=[
          pltpu.VMEM((2, _N, _D), jnp.float32),
          pltpu.SemaphoreType.DMA((2,)),
      ],
      compiler_params=pltpu.CompilerParams(
          dimension_semantics=("arbitrary",)),
  )(wl3, u03, particles, w1t, w2r)

  return est3.reshape(_B, _D)


# speedup vs baseline: 1.2888x; 1.1827x over previous
import jax
import jax.numpy as jnp
from jax.experimental import pallas as pl
from jax.experimental.pallas import tpu as pltpu

_B, _N, _D, _H = 128, 8192, 64, 256
_ALPHA = 0.5
_CN = 4096


def _cumsum_last(x):
  n = x.shape[-1]
  d = 1
  while d < n:
    x = x + jnp.concatenate([jnp.zeros_like(x[..., :d]), x[..., :-d]], axis=-1)
    d *= 2
  return x


def _body(wl_ref, u0_ref, p_ref, w1t_ref, w2_ref, o_ref):
  wl = wl_ref[0]
  u0 = u0_ref[0]
  m = jnp.max(wl, axis=-1, keepdims=True)
  e = jnp.exp(wl - m)
  s = jnp.sum(e, axis=-1, keepdims=True)
  probs = e / s
  soft = _ALPHA * probs + (1.0 - _ALPHA) / _N
  cum = _cumsum_last(soft)
  cum = cum / cum[..., -1:]
  t = cum * _N
  k = jnp.floor(t)
  g = k + jnp.where(k + u0 <= t, 1.0, 0.0)
  g = jnp.clip(g, 0.0, float(_N))
  gprev = jnp.concatenate([jnp.zeros_like(g[..., :1]), g[..., :-1]], axis=-1)
  count = g - gprev
  a = count * (probs / (soft + 1e-8) + 1e-8)

  w1tb = w1t_ref[...].astype(jnp.bfloat16)
  est = jnp.zeros((1, _D), jnp.float32)
  zsum = jnp.zeros((1, 1), jnp.float32)
  for c in range(_N // _CN):
    p = p_ref[0, c * _CN:(c + 1) * _CN, :]
    z = jax.lax.dot_general(w1tb, p.astype(jnp.bfloat16),
                            (((1,), (1,)), ((), ())),
                            preferred_element_type=jnp.float32)
    h = jnp.tanh(z).astype(jnp.bfloat16)
    lik = jnp.dot(w2_ref[...].astype(jnp.bfloat16), h,
                  preferred_element_type=jnp.float32)
    t_ = a[:, c * _CN:(c + 1) * _CN] * jnp.exp(lik)
    est += jnp.dot(t_, p, preferred_element_type=jnp.float32)
    zsum += jnp.sum(t_, axis=-1, keepdims=True)
  o_ref[...] = (est / zsum)[None]


def kernel(particles, weights_log, u0, W1, w2):
  wl3 = weights_log.reshape(_B, 1, _N)
  u03 = u0.reshape(_B, 1, 1)
  w1t = W1.T
  w2r = w2.reshape(1, _H)

  est3 = pl.pallas_call(
      _body,
      grid=(_B,),
      in_specs=[
          pl.BlockSpec((1, 1, _N), lambda b: (b, 0, 0)),
          pl.BlockSpec((1, 1, 1), lambda b: (b, 0, 0)),
          pl.BlockSpec((1, _N, _D), lambda b: (b, 0, 0)),
          pl.BlockSpec((_H, _D), lambda b: (0, 0)),
          pl.BlockSpec((1, _H), lambda b: (0, 0)),
      ],
      out_specs=pl.BlockSpec((1, 1, _D), lambda b: (b, 0, 0)),
      out_shape=jax.ShapeDtypeStruct((_B, 1, _D), jnp.float32),
      compiler_params=pltpu.CompilerParams(
          dimension_semantics=("parallel",)),
  )(wl3, u03, particles, w1t, w2r)

  return est3.reshape(_B, _D)
